# baseline (speedup 1.0000x reference)
import functools

import jax
import jax.numpy as jnp
from jax import lax
from jax.experimental import pallas as pl
from jax.experimental.pallas import tpu as pltpu
from jax.experimental.pallas import tpu_sc as plsc

IN_FEATURES = 4096
OUT_FEATURES = 4096
GROUPS = 32
PACK = 8
MAXQ = 15
L = 16
NC = 2
NS = 16
NW = NC * NS
RPW = IN_FEATURES // NW
PPW = RPW // PACK
PZ = OUT_FEATURES // PACK
KV = OUT_FEATURES // L


@functools.partial(
    pl.kernel,
    mesh=plsc.VectorSubcoreMesh(core_axis_name="c", subcore_axis_name="s"),
    out_type=jax.ShapeDtypeStruct((IN_FEATURES * OUT_FEATURES,), jnp.float32),
    scratch_types=[
        pltpu.VMEM((RPW + PACK,), jnp.int32),
        pltpu.VMEM((OUT_FEATURES,), jnp.float32),
        pltpu.VMEM((OUT_FEATURES,), jnp.float32),
        pltpu.VMEM((PZ,), jnp.int32),
        pltpu.VMEM((OUT_FEATURES,), jnp.int32),
        pltpu.VMEM((PACK * OUT_FEATURES,), jnp.float32),
    ],
)
def _dequant(qw_hbm, qz_hbm, sc_hbm, g_hbm, out_hbm,
             g_v, sc_v, z_v, qz_v, qw_v, out_v):
    wid = lax.axis_index("s") * NC + lax.axis_index("c")
    prow0 = wid * PPW
    row0 = wid * RPW
    iota = lax.iota(jnp.int32, L)
    zshift = 4 * (iota & 7)
    pidx = iota >> 3

    pltpu.sync_copy(g_hbm.at[pl.ds(row0, RPW)], g_v.at[pl.ds(0, RPW)])

    def refill(gr):
        pltpu.sync_copy(sc_hbm.at[pl.ds(gr * OUT_FEATURES, OUT_FEATURES)], sc_v)
        pltpu.sync_copy(qz_hbm.at[pl.ds(gr * PZ, PZ)], qz_v)

        def zm(m, c):
            win = qz_v[pl.ds(m * L, L)]
            for t in range(PACK):
                w0 = win[2 * t]
                w1 = win[2 * t + 1]
                qzv = jnp.where(iota < PACK, w0, w1)
                z = ((qzv >> zshift) & MAXQ).astype(jnp.float32)
                z_v[pl.ds((m * PACK + t) * L, L)] = z
            return c

        lax.fori_loop(0, PZ // L, zm, 0)
        return gr

    def maybe_refill(gr, gc):
        return lax.cond(gr != gc, lambda: refill(gr), lambda: gc)

    def emit_row(j, base):
        def fk(k, c):
            qv = qw_v[pl.ds(k * L, L)]
            wf = ((qv >> (4 * j)) & MAXQ).astype(jnp.float32)
            sv = sc_v[pl.ds(k * L, L)]
            zv = z_v[pl.ds(k * L, L)]
            out_v[pl.ds(base + k * L, L)] = sv * (wf - zv)
            return c

        lax.fori_loop(0, KV, fk, 0)

    def fast_block(p):
        def fk(k, c):
            qv = qw_v[pl.ds(k * L, L)]
            sv = sc_v[pl.ds(k * L, L)]
            zv = z_v[pl.ds(k * L, L)]
            for j in range(PACK):
                wf = ((qv >> (4 * j)) & MAXQ).astype(jnp.float32)
                out_v[pl.ds(j * OUT_FEATURES + k * L, L)] = sv * (wf - zv)
            return c

        lax.fori_loop(0, KV, fk, 0)

    def prow(p, gc):
        pltpu.sync_copy(qw_hbm.at[pl.ds((prow0 + p) * OUT_FEATURES, OUT_FEATURES)],
                        qw_v)
        gwin = g_v[pl.ds(p * PACK, L)]
        g0 = gwin[0]
        g7 = gwin[PACK - 1]

        def uniform(gc):
            gc = maybe_refill(g0, gc)
            fast_block(p)
            return gc

        def mixed(gc):
            for j in range(PACK):
                gc = maybe_refill(gwin[j], gc)
                emit_row(j, j * OUT_FEATURES)
            return gc

        gc = lax.cond(g0 == g7, uniform, mixed, gc)
        pltpu.sync_copy(out_v,
                        out_hbm.at[pl.ds((row0 + p * PACK) * OUT_FEATURES,
                                         PACK * OUT_FEATURES)])
        return gc

    lax.fori_loop(0, PPW, prow, jnp.int32(-1))


def kernel(qweight, qzeros, scales, g_idx, num_itr=1):
    g = g_idx.astype(jnp.int32) + (jnp.asarray(num_itr, jnp.int32) - 1)
    g = jnp.clip(g, 0, GROUPS - 1)
    out = _dequant(qweight.reshape(-1), qzeros.reshape(-1), scales.reshape(-1), g)
    return out.reshape(IN_FEATURES, OUT_FEATURES)


# speedup vs baseline: 1.4222x; 1.4222x over previous
import functools

import jax
import jax.numpy as jnp
from jax import lax
from jax.experimental import pallas as pl
from jax.experimental.pallas import tpu as pltpu
from jax.experimental.pallas import tpu_sc as plsc

IN_FEATURES = 4096
OUT_FEATURES = 4096
GROUPS = 32
PACK = 8
MAXQ = 15
L = 16
NC = 2
NS = 16
NW = NC * NS
RPW = IN_FEATURES // NW
PPW = RPW // PACK
PZ = OUT_FEATURES // PACK
KV = OUT_FEATURES // L


@functools.partial(
    pl.kernel,
    mesh=plsc.VectorSubcoreMesh(core_axis_name="c", subcore_axis_name="s"),
    out_type=jax.ShapeDtypeStruct((IN_FEATURES * OUT_FEATURES,), jnp.float32),
    scratch_types=[
        pltpu.VMEM((RPW + PACK,), jnp.int32),
        pltpu.VMEM((OUT_FEATURES,), jnp.float32),
        pltpu.VMEM((OUT_FEATURES,), jnp.float32),
        pltpu.VMEM((PZ,), jnp.int32),
        pltpu.VMEM((OUT_FEATURES,), jnp.int32),
        pltpu.VMEM((PACK * OUT_FEATURES,), jnp.float32),
    ],
)
def _dequant(qw_hbm, qz_hbm, sc_hbm, g_hbm, out_hbm,
             g_v, sc_v, z_v, qz_v, qw_v, out_v):
    wid = lax.axis_index("s") * NC + lax.axis_index("c")
    prow0 = wid * PPW
    row0 = wid * RPW
    iota = lax.iota(jnp.int32, L)
    zshift = 4 * (iota & 7)
    pidx = iota >> 3

    pltpu.sync_copy(g_hbm.at[pl.ds(row0, RPW)], g_v.at[pl.ds(0, RPW)])

    def refill(gr):
        pltpu.sync_copy(sc_hbm.at[pl.ds(gr * OUT_FEATURES, OUT_FEATURES)], sc_v)
        pltpu.sync_copy(qz_hbm.at[pl.ds(gr * PZ, PZ)], qz_v)

        def zm(m, c):
            win = qz_v[pl.ds(m * L, L)]
            for t in range(PACK):
                w0 = win[2 * t]
                w1 = win[2 * t + 1]
                qzv = jnp.where(iota < PACK, w0, w1)
                z = ((qzv >> zshift) & MAXQ).astype(jnp.float32)
                z_v[pl.ds((m * PACK + t) * L, L)] = z
            return c

        lax.fori_loop(0, PZ // L, zm, 0)
        return gr

    def maybe_refill(gr, gc):
        return lax.cond(gr != gc, lambda: refill(gr), lambda: gc)

    def tile_base(k):
        return 896 * (k // 8) + 16 * k

    def emit_row(j):
        def fk(k, c):
            qv = qw_v[pl.ds(k * L, L)]
            wf = ((qv >> (4 * j)) & MAXQ).astype(jnp.float32)
            sv = sc_v[pl.ds(k * L, L)]
            zv = z_v[pl.ds(k * L, L)]
            out_v[pl.ds(tile_base(k) + 128 * j, L)] = sv * (wf - zv)
            return c

        lax.fori_loop(0, KV, fk, 0)

    def fast_block(p):
        def fk(k, c):
            qv = qw_v[pl.ds(k * L, L)]
            sv = sc_v[pl.ds(k * L, L)]
            zv = z_v[pl.ds(k * L, L)]
            base = tile_base(k)
            for j in range(PACK):
                wf = ((qv >> (4 * j)) & MAXQ).astype(jnp.float32)
                out_v[pl.ds(base + 128 * j, L)] = sv * (wf - zv)
            return c

        lax.fori_loop(0, KV, fk, 0)

    def prow(p, gc):
        pltpu.sync_copy(qw_hbm.at[pl.ds((prow0 + p) * OUT_FEATURES, OUT_FEATURES)],
                        qw_v)
        gwin = g_v[pl.ds(p * PACK, L)]
        g0 = gwin[0]
        g7 = gwin[PACK - 1]

        def uniform(gc):
            gc = maybe_refill(g0, gc)
            fast_block(p)
            return gc

        def mixed(gc):
            for j in range(PACK):
                gc = maybe_refill(gwin[j], gc)
                emit_row(j)
            return gc

        gc = lax.cond(g0 == g7, uniform, mixed, gc)
        pltpu.sync_copy(out_v,
                        out_hbm.at[pl.ds((row0 + p * PACK) * OUT_FEATURES,
                                         PACK * OUT_FEATURES)])
        return gc

    lax.fori_loop(0, PPW, prow, jnp.int32(-1))


def kernel(qweight, qzeros, scales, g_idx, num_itr=1):
    g = g_idx.astype(jnp.int32) + (jnp.asarray(num_itr, jnp.int32) - 1)
    g = jnp.clip(g, 0, GROUPS - 1)
    out = _dequant(qweight.reshape(-1), qzeros.reshape(-1), scales.reshape(-1), g)
    out = out.reshape(IN_FEATURES // PACK, OUT_FEATURES // 128, PACK, 128)
    out = out.transpose(0, 2, 1, 3)
    return out.reshape(IN_FEATURES, OUT_FEATURES)


# speedup vs baseline: 1.7897x; 1.2584x over previous
import functools

import jax
import jax.numpy as jnp
from jax import lax
from jax.experimental import pallas as pl
from jax.experimental.pallas import tpu as pltpu
from jax.experimental.pallas import tpu_sc as plsc

IN_FEATURES = 4096
OUT_FEATURES = 4096
GROUPS = 32
PACK = 8
MAXQ = 15
L = 16
NC = 2
NS = 16
NW = NC * NS
RPW = IN_FEATURES // NW
PPW = RPW // PACK
PZ = OUT_FEATURES // PACK
KV = OUT_FEATURES // L


@functools.partial(
    pl.kernel,
    mesh=plsc.VectorSubcoreMesh(core_axis_name="c", subcore_axis_name="s"),
    out_type=jax.ShapeDtypeStruct((IN_FEATURES * OUT_FEATURES,), jnp.float32),
    scratch_types=[
        pltpu.VMEM((RPW + PACK,), jnp.int32),
        pltpu.VMEM((OUT_FEATURES,), jnp.float32),
        pltpu.VMEM((OUT_FEATURES,), jnp.float32),
        pltpu.VMEM((PZ,), jnp.int32),
        pltpu.VMEM((OUT_FEATURES,), jnp.int32),
        pltpu.VMEM((OUT_FEATURES,), jnp.int32),
        pltpu.VMEM((PACK * OUT_FEATURES,), jnp.float32),
        pltpu.VMEM((PACK * OUT_FEATURES,), jnp.float32),
        pltpu.SemaphoreType.DMA,
        pltpu.SemaphoreType.DMA,
        pltpu.SemaphoreType.DMA,
        pltpu.SemaphoreType.DMA,
    ],
)
def _dequant(qw_hbm, qz_hbm, sc_hbm, g_hbm, out_hbm,
             g_v, sc_v, z_v, qz_v, qw0, qw1, ov0, ov1, si0, si1, so0, so1):
    wid = lax.axis_index("s") * NC + lax.axis_index("c")
    prow0 = wid * PPW
    row0 = wid * RPW
    iota = lax.iota(jnp.int32, L)
    zshift = 4 * (iota & 7)
    pidx = iota >> 3

    pltpu.sync_copy(g_hbm.at[pl.ds(row0, RPW)], g_v.at[pl.ds(0, RPW)])

    def refill(gr):
        pltpu.sync_copy(sc_hbm.at[pl.ds(gr * OUT_FEATURES, OUT_FEATURES)], sc_v)
        pltpu.sync_copy(qz_hbm.at[pl.ds(gr * PZ, PZ)], qz_v)

        def zm(m, c):
            win = qz_v[pl.ds(m * L, L)]
            for t in range(PACK):
                w0 = win[2 * t]
                w1 = win[2 * t + 1]
                qzv = jnp.where(iota < PACK, w0, w1)
                z = ((qzv >> zshift) & MAXQ).astype(jnp.float32)
                z_v[pl.ds((m * PACK + t) * L, L)] = z
            return c

        lax.fori_loop(0, PZ // L, zm, 0)
        return gr

    def maybe_refill(gr, gc):
        return lax.cond(gr != gc, lambda: refill(gr), lambda: gc)

    def tile_base(k):
        return 896 * (k // 8) + 16 * k

    def emit_row(j, qw_v, out_v):
        def fk(k, c):
            qv = qw_v[pl.ds(k * L, L)]
            wf = ((qv >> (4 * j)) & MAXQ).astype(jnp.float32)
            sv = sc_v[pl.ds(k * L, L)]
            zv = z_v[pl.ds(k * L, L)]
            out_v[pl.ds(tile_base(k) + 128 * j, L)] = sv * (wf - zv)
            return c

        lax.fori_loop(0, KV, fk, 0)

    def fast_block(qw_v, out_v):
        def fk(k, c):
            qv = qw_v[pl.ds(k * L, L)]
            sv = sc_v[pl.ds(k * L, L)]
            zv = z_v[pl.ds(k * L, L)]
            base = tile_base(k)
            for j in range(PACK):
                wf = ((qv >> (4 * j)) & MAXQ).astype(jnp.float32)
                out_v[pl.ds(base + 128 * j, L)] = sv * (wf - zv)
            return c

        lax.fori_loop(0, KV, fk, 0)

    qwb, ovb, sib, sob = (qw0, qw1), (ov0, ov1), (si0, si1), (so0, so1)

    def qw_src(p):
        return qw_hbm.at[pl.ds((prow0 + p) * OUT_FEATURES, OUT_FEATURES)]

    def out_dst(p):
        return out_hbm.at[pl.ds((row0 + p * PACK) * OUT_FEATURES,
                                PACK * OUT_FEATURES)]

    pltpu.async_copy(qw_src(0), qw0, si0)
    pltpu.async_copy(qw_src(1), qw1, si1)

    def pair(q, gc):
        for b in range(2):
            p = 2 * q + b
            pltpu.make_async_copy(qw_src(0), qwb[b], sib[b]).wait()

            @pl.when(q > 0)
            def _wait_out():
                pltpu.make_async_copy(ovb[b], out_dst(0), sob[b]).wait()

            gwin = g_v[pl.ds(p * PACK, L)]
            g0 = gwin[0]
            g7 = gwin[PACK - 1]

            def uniform(gc, b=b, g0=g0):
                gc = maybe_refill(g0, gc)
                fast_block(qwb[b], ovb[b])
                return gc

            def mixed(gc, b=b, gwin=gwin):
                for j in range(PACK):
                    gc = maybe_refill(gwin[j], gc)
                    emit_row(j, qwb[b], ovb[b])
                return gc

            gc = lax.cond(g0 == g7, uniform, mixed, gc)
            pltpu.async_copy(ovb[b], out_dst(p), sob[b])

            @pl.when(p + 2 < PPW)
            def _next_qw():
                pltpu.async_copy(qw_src(p + 2), qwb[b], sib[b])

        return gc

    lax.fori_loop(0, PPW // 2, pair, jnp.int32(-1))
    pltpu.make_async_copy(ov0, out_dst(0), so0).wait()
    pltpu.make_async_copy(ov1, out_dst(0), so1).wait()


def kernel(qweight, qzeros, scales, g_idx, num_itr=1):
    g = g_idx.astype(jnp.int32) + (jnp.asarray(num_itr, jnp.int32) - 1)
    g = jnp.clip(g, 0, GROUPS - 1)
    out = _dequant(qweight.reshape(-1), qzeros.reshape(-1), scales.reshape(-1), g)
    out = out.reshape(IN_FEATURES // PACK, OUT_FEATURES // 128, PACK, 128)
    out = out.transpose(0, 2, 1, 3)
    return out.reshape(IN_FEATURES, OUT_FEATURES)
